# baseline (speedup 1.0000x reference)
import functools

import jax
import jax.numpy as jnp
from jax import lax
from jax.experimental import pallas as pl
from jax.experimental.pallas import tpu as pltpu
from jax.experimental.pallas import tpu_sc as plsc

N_CTX = 8
CTX_LEN = 77
CTX_DIM = 768
BATCH = 1024
NUM_WORKERS = 32
B_PER_W = BATCH // NUM_WORKERS
N_SUF = CTX_LEN - 1 - N_CTX
SUF0 = 1 + N_CTX
N_GAT = 1 + N_SUF


def _prompt_body(tok_hbm, idx_hbm, table_hbm, ctxp_hbm, ctxn_hbm,
                 out_hbm, tokout_hbm,
                 tokblk_v, idxblk_v, buf0, buf1, ctxp_v, ctxn_v,
                 sem_g0, sem_g1, sem_w0, sem_w1, sem_c):
    wid = lax.axis_index("s") * 2 + lax.axis_index("c")
    base = wid * B_PER_W

    pltpu.sync_copy(ctxp_hbm.at[0], ctxp_v)
    pltpu.sync_copy(ctxn_hbm.at[0], ctxn_v)
    pltpu.sync_copy(tok_hbm.at[pl.ds(base, B_PER_W)], tokblk_v)
    pltpu.sync_copy(idx_hbm.at[pl.ds(base, B_PER_W)], idxblk_v)

    pltpu.sync_copy(tokblk_v, tokout_hbm.at[pl.ds(base, B_PER_W)])
    pltpu.sync_copy(tokblk_v, tokout_hbm.at[pl.ds(base + BATCH, B_PER_W)])

    ctx_w = []
    for li in range(B_PER_W):
        b = base + li
        ctx_w.append(pltpu.async_copy(
            ctxp_v, out_hbm.at[b, pl.ds(1, N_CTX)], sem_c))
        ctx_w.append(pltpu.async_copy(
            ctxn_v, out_hbm.at[b + BATCH, pl.ds(1, N_CTX)], sem_c))

    def gather(li, buf, sem):
        return (pltpu.async_copy(table_hbm.at[idxblk_v.at[li]], buf, sem),)

    def write(b, buf, sem):
        return (
            pltpu.async_copy(buf.at[pl.ds(0, 1)],
                             out_hbm.at[b, pl.ds(0, 1)], sem),
            pltpu.async_copy(buf.at[pl.ds(1, N_SUF)],
                             out_hbm.at[b, pl.ds(SUF0, N_SUF)], sem),
            pltpu.async_copy(buf.at[pl.ds(0, 1)],
                             out_hbm.at[b + BATCH, pl.ds(0, 1)], sem),
            pltpu.async_copy(buf.at[pl.ds(1, N_SUF)],
                             out_hbm.at[b + BATCH, pl.ds(SUF0, N_SUF)], sem),
        )

    def wait(descrs):
        for d in descrs:
            d.wait()

    bufs = (buf0, buf1)
    gsems = (sem_g0, sem_g1)
    wsems = (sem_w0, sem_w1)
    pend_w = [None, None]

    g = gather(0, bufs[0], gsems[0])
    for li in range(B_PER_W):
        cur = li & 1
        nxt = 1 - cur
        wait(g)
        if li + 1 < B_PER_W:
            if pend_w[nxt] is not None:
                wait(pend_w[nxt])
                pend_w[nxt] = None
            g = gather(li + 1, bufs[nxt], gsems[nxt])
        pend_w[cur] = write(base + li, bufs[cur], wsems[cur])
    for p in pend_w:
        if p is not None:
            wait(p)
    wait(ctx_w)


def kernel(tokenized_prompts, token_embedding, ctx_pos, ctx_neg):
    mesh = plsc.VectorSubcoreMesh(core_axis_name="c", subcore_axis_name="s")
    f = functools.partial(
        pl.kernel,
        mesh=mesh,
        compiler_params=pltpu.CompilerParams(use_tc_tiling_on_sc=False),
        out_type=(
            jax.ShapeDtypeStruct((2 * BATCH, CTX_LEN, CTX_DIM), jnp.float32),
            jax.ShapeDtypeStruct((2 * BATCH, CTX_LEN), jnp.int32),
        ),
        scratch_types=[
            pltpu.VMEM((B_PER_W, CTX_LEN), jnp.int32),
            pltpu.VMEM((B_PER_W, N_GAT), jnp.int32),
            pltpu.VMEM((N_GAT, CTX_DIM), jnp.float32),
            pltpu.VMEM((N_GAT, CTX_DIM), jnp.float32),
            pltpu.VMEM((N_CTX, CTX_DIM), jnp.float32),
            pltpu.VMEM((N_CTX, CTX_DIM), jnp.float32),
            pltpu.SemaphoreType.DMA,
            pltpu.SemaphoreType.DMA,
            pltpu.SemaphoreType.DMA,
            pltpu.SemaphoreType.DMA,
            pltpu.SemaphoreType.DMA,
        ],
    )(_prompt_body)
    idx_gat = jnp.concatenate(
        [tokenized_prompts[:, :1], tokenized_prompts[:, SUF0:]], axis=1)
    return f(tokenized_prompts, idx_gat, token_embedding, ctx_pos, ctx_neg)
